# baseline (speedup 1.0000x reference)
import functools

import jax
import jax.numpy as jnp
from jax import lax
from jax.experimental import pallas as pl
from jax.experimental.pallas import tpu as pltpu
from jax.experimental.pallas import tpu_sc as plsc

N_USERS = 1_000_000
N_ITEMS = 1_000_000
N_LATENT = 16
N_REVIEWS = 1_048_576

NC = 2
NS = 16
L = 16
NW = NC * NS
PER_W = N_REVIEWS // NW
CHUNK = 1024
N_CHUNKS = PER_W // CHUNK
SUB = 64
N_SUB = CHUNK // SUB
N_GROUPS = CHUNK // L


NBLK = N_USERS // 128
TAIL = N_USERS - NBLK * 128
KB = 12
NSUP = NBLK // KB
SUPW = KB * 128
SUPO = SUPW * N_LATENT


def _untangle_body(tht_hbm, xt_hbm, th_tail_hbm, x_tail_hbm,
                   thf_hbm, xf_hbm,
                   tiles_v, out_v, sem_in, sem_out):
    cid = lax.axis_index("c")
    sid = lax.axis_index("s")
    wid = cid * NS + sid
    lo = wid * (NSUP // NW) + jnp.minimum(wid, NSUP % NW)
    nm = (NSUP // NW) + (wid < NSUP % NW)

    lanes16 = lax.iota(jnp.int32, L) * N_LATENT

    def do_table(src_hbm, dst_hbm):
        def issue_in(j):
            u0 = (lo + j) * SUPW
            srow = (j % 2) * L
            for rb in range(2):
                pltpu.async_copy(
                    src_hbm.at[pl.ds(8 * rb, 8), pl.ds(u0, SUPW)],
                    tiles_v.at[pl.ds(srow + rb * 8, 8), :], sem_in)

        def wait_in():
            for rb in range(2):
                pltpu.make_async_copy(
                    src_hbm.at[pl.ds(0, 8), pl.ds(0, SUPW)],
                    tiles_v.at[pl.ds(0, 8), :], sem_in).wait()

        issue_in(0)

        def sup(j, carry):
            s = j % 2
            wait_in()

            @pl.when(j + 1 < nm)
            def _():
                issue_in(j + 1)

            @pl.when(j >= 2)
            def _():
                pltpu.make_async_copy(
                    out_v.at[pl.ds(0, SUPO)], dst_hbm.at[pl.ds(0, SUPO)],
                    sem_out).wait()

            obase = s * SUPO

            def blk(b, c2):
                cbase = b * 128
                for d in range(N_LATENT):
                    rb, r = divmod(d, 8)
                    for o in range(0, 128, L):
                        v = tiles_v[s * L + rb * 8 + r,
                                    pl.ds(cbase + o, L)]
                        idx = lanes16 + (
                            obase + cbase * N_LATENT + o * N_LATENT + d)
                        plsc.store_scatter(out_v, [idx], v)
                return c2

            lax.fori_loop(0, KB, blk, 0)
            pltpu.async_copy(
                out_v.at[pl.ds(obase, SUPO)],
                dst_hbm.at[pl.ds((lo + j) * SUPO, SUPO)],
                sem_out)
            return carry

        lax.fori_loop(0, nm, sup, 0)
        lax.fori_loop(
            0, jnp.minimum(nm, 2),
            lambda i, c: (pltpu.make_async_copy(
                out_v.at[pl.ds(0, SUPO)], dst_hbm.at[pl.ds(0, SUPO)],
                sem_out).wait(), c)[1],
            0)

    do_table(tht_hbm, thf_hbm)
    do_table(xt_hbm, xf_hbm)

    @pl.when(wid == NW - 1)
    def _():
        for tail_hbm, dst_hbm in ((th_tail_hbm, thf_hbm),
                                  (x_tail_hbm, xf_hbm)):
            sl = pl.ds(0, TAIL * N_LATENT)
            pltpu.sync_copy(tail_hbm, out_v.at[sl])
            pltpu.sync_copy(
                out_v.at[sl],
                dst_hbm.at[pl.ds(NBLK * 2048, TAIL * N_LATENT)])


def _body(theta_hbm, x_hbm, tgt_hbm, uidx_hbm, iidx_hbm,
          pred_hbm, loss_hbm,
          uidx_v, iidx_v, tgt_v, th_v, x_v, pred_v, stage_v,
          lpart_v, lred_v, loss_shared, sem_idx, sem_rows, sem_out):
    cid = lax.axis_index("c")
    sid = lax.axis_index("s")
    wid = cid * NS + sid
    base_w = wid * PER_W

    lanes_x16 = lax.iota(jnp.int32, L) * L

    def issue_idx(k, off):
        start = base_w + k * CHUNK
        sl = pl.ds(off, CHUNK)
        pltpu.async_copy(uidx_hbm.at[pl.ds(start, CHUNK)], uidx_v.at[sl],
                         sem_idx)
        pltpu.async_copy(iidx_hbm.at[pl.ds(start, CHUNK)], iidx_v.at[sl],
                         sem_idx)
        pltpu.async_copy(tgt_hbm.at[pl.ds(start, CHUNK)], tgt_v.at[sl],
                         sem_idx)

    def wait_idx():
        for src, ref in ((uidx_hbm, uidx_v), (iidx_hbm, iidx_v),
                         (tgt_hbm, tgt_v)):
            pltpu.make_async_copy(
                src.at[pl.ds(0, CHUNK)], ref.at[pl.ds(0, CHUNK)],
                sem_idx).wait()

    def issue_gathers(off):
        for j in range(N_SUB):
            sl = pl.ds(off + j * SUB, SUB)
            pltpu.async_copy(theta_hbm.at[uidx_v.at[sl]], th_v.at[sl],
                             sem_rows)
            pltpu.async_copy(x_hbm.at[iidx_v.at[sl]], x_v.at[sl],
                             sem_rows)

    def wait_gathers():
        for ref in (th_v, x_v):
            pltpu.make_async_copy(
                theta_hbm.at[pl.ds(0, CHUNK)], ref.at[pl.ds(0, CHUNK)],
                sem_rows).wait()

    issue_idx(0, 0)
    wait_idx()
    issue_gathers(0)
    issue_idx(1, CHUNK)

    def chunk_body(k, loss_acc):
        off = (k % 2) * CHUNK
        start = base_w + k * CHUNK
        wait_gathers()

        @pl.when(k + 1 < N_CHUNKS)
        def _():
            wait_idx()
            issue_gathers((k + 1) % 2 * CHUNK)

        @pl.when(k >= 2)
        def _():
            pltpu.make_async_copy(
                pred_v.at[pl.ds(0, CHUNK)], pred_hbm.at[pl.ds(0, CHUNK)],
                sem_out).wait()

        def group_body(g, acc):
            rbase = off + g * L
            sbase = (g % 2) * (L * L)
            for i in range(L):
                r = rbase + i
                prod = th_v[r] * x_v[r]
                plsc.store_scatter(stage_v, [lanes_x16 + (sbase + i)], prod)
            p = stage_v[pl.ds(sbase, L)]
            for d in range(1, L):
                p = p + stage_v[pl.ds(sbase + d * L, L)]
            pred_v[pl.ds(rbase, L)] = p
            diff = p - tgt_v[pl.ds(rbase, L)]
            return acc + diff * diff

        loss_acc = lax.fori_loop(0, N_GROUPS, group_body, loss_acc)
        pltpu.async_copy(pred_v.at[pl.ds(off, CHUNK)],
                         pred_hbm.at[pl.ds(start, CHUNK)], sem_out)

        @pl.when(k + 2 < N_CHUNKS)
        def _():
            issue_idx(k + 2, off)

        return loss_acc

    loss_acc = lax.fori_loop(
        0, N_CHUNKS, chunk_body, jnp.zeros((L,), jnp.float32))

    for _ in range(2):
        pltpu.make_async_copy(
            pred_v.at[pl.ds(0, CHUNK)], pred_hbm.at[pl.ds(0, CHUNK)],
            sem_out).wait()

    lpart_v[...] = loss_acc
    pltpu.sync_copy(lpart_v, loss_shared.at[sid])
    plsc.subcore_barrier()

    @pl.when(sid == 0)
    def _():
        pltpu.sync_copy(loss_shared, lred_v)
        tot = lred_v[0]
        for s in range(1, NS):
            tot = tot + lred_v[s]
        scalar = jnp.sum(tot) * (1.0 / float(N_REVIEWS))
        lpart_v[...] = jnp.broadcast_to(scalar, (L,))
        pltpu.sync_copy(lpart_v, loss_hbm.at[cid])


@jax.jit
def _mf_forward(theta, X, targets, uidx, iidx):
    mesh = plsc.VectorSubcoreMesh(
        core_axis_name="c", subcore_axis_name="s",
        num_cores=NC, num_subcores=NS)
    untangle = pl.kernel(
        _untangle_body,
        out_type=(
            jax.ShapeDtypeStruct((N_USERS * N_LATENT,), jnp.float32),
            jax.ShapeDtypeStruct((N_ITEMS * N_LATENT,), jnp.float32),
        ),
        mesh=mesh,
        scratch_types=[
            pltpu.VMEM((2 * L, SUPW), jnp.float32),
            pltpu.VMEM((2 * SUPO,), jnp.float32),
            pltpu.SemaphoreType.DMA,
            pltpu.SemaphoreType.DMA,
        ],
        compiler_params=pltpu.CompilerParams(
            needs_layout_passes=False, use_tc_tiling_on_sc=True),
    )
    th_tail = theta[NBLK * 128:].reshape(-1)
    x_tail = X[NBLK * 128:].reshape(-1)
    th_flat, x_flat = untangle(theta.T, X.T, th_tail, x_tail)
    theta_rm = th_flat.reshape(N_USERS, N_LATENT)
    x_rm = x_flat.reshape(N_ITEMS, N_LATENT)
    f = pl.kernel(
        _body,
        out_type=(
            jax.ShapeDtypeStruct((N_REVIEWS,), jnp.float32),
            jax.ShapeDtypeStruct((NC, L), jnp.float32),
        ),
        mesh=mesh,
        scratch_types=[
            pltpu.VMEM((2 * CHUNK,), jnp.int32),
            pltpu.VMEM((2 * CHUNK,), jnp.int32),
            pltpu.VMEM((2 * CHUNK,), jnp.float32),
            pltpu.VMEM((2 * CHUNK, N_LATENT), jnp.float32),
            pltpu.VMEM((2 * CHUNK, N_LATENT), jnp.float32),
            pltpu.VMEM((2 * CHUNK,), jnp.float32),
            pltpu.VMEM((2 * L * L,), jnp.float32),
            pltpu.VMEM((L,), jnp.float32),
            pltpu.VMEM((NS, L), jnp.float32),
            pltpu.VMEM_SHARED((NS, L), jnp.float32),
            pltpu.SemaphoreType.DMA,
            pltpu.SemaphoreType.DMA,
            pltpu.SemaphoreType.DMA,
        ],
        compiler_params=pltpu.CompilerParams(
            needs_layout_passes=False, use_tc_tiling_on_sc=False),
    )
    return f(theta_rm, x_rm, targets, uidx, iidx)


def kernel(theta, X, targets, user_indices, item_indices):
    uidx = user_indices.astype(jnp.int32)
    iidx = item_indices.astype(jnp.int32)
    pred, loss_parts = _mf_forward(theta, X, targets, uidx, iidx)
    loss = loss_parts[0, 0] + loss_parts[1, 0]
    return (pred, loss)


# speedup vs baseline: 1.0014x; 1.0014x over previous
import functools

import jax
import jax.numpy as jnp
from jax import lax
from jax.experimental import pallas as pl
from jax.experimental.pallas import tpu as pltpu
from jax.experimental.pallas import tpu_sc as plsc

N_USERS = 1_000_000
N_ITEMS = 1_000_000
N_LATENT = 16
N_REVIEWS = 1_048_576

NC = 2
NS = 16
L = 16
NW = NC * NS
PER_W = N_REVIEWS // NW
CHUNK = 1024
N_CHUNKS = PER_W // CHUNK
SUB = 128
N_SUB = CHUNK // SUB
N_GROUPS = CHUNK // L


NBLK = N_USERS // 128
TAIL = N_USERS - NBLK * 128
KB = 12
NSUP = NBLK // KB
SUPW = KB * 128
SUPO = SUPW * N_LATENT


def _untangle_body(tht_hbm, xt_hbm, th_tail_hbm, x_tail_hbm,
                   thf_hbm, xf_hbm,
                   tiles_v, out_v, sem_in, sem_out):
    cid = lax.axis_index("c")
    sid = lax.axis_index("s")
    wid = cid * NS + sid
    lo = wid * (NSUP // NW) + jnp.minimum(wid, NSUP % NW)
    nm = (NSUP // NW) + (wid < NSUP % NW)

    lanes16 = lax.iota(jnp.int32, L) * N_LATENT

    def do_table(src_hbm, dst_hbm):
        def issue_in(j):
            u0 = (lo + j) * SUPW
            srow = (j % 2) * L
            for rb in range(2):
                pltpu.async_copy(
                    src_hbm.at[pl.ds(8 * rb, 8), pl.ds(u0, SUPW)],
                    tiles_v.at[pl.ds(srow + rb * 8, 8), :], sem_in)

        def wait_in():
            for rb in range(2):
                pltpu.make_async_copy(
                    src_hbm.at[pl.ds(0, 8), pl.ds(0, SUPW)],
                    tiles_v.at[pl.ds(0, 8), :], sem_in).wait()

        issue_in(0)

        def sup(j, carry):
            s = j % 2
            wait_in()

            @pl.when(j + 1 < nm)
            def _():
                issue_in(j + 1)

            @pl.when(j >= 2)
            def _():
                pltpu.make_async_copy(
                    out_v.at[pl.ds(0, SUPO)], dst_hbm.at[pl.ds(0, SUPO)],
                    sem_out).wait()

            obase = s * SUPO

            def blk(b, c2):
                cbase = b * 128
                for d in range(N_LATENT):
                    rb, r = divmod(d, 8)
                    for o in range(0, 128, L):
                        v = tiles_v[s * L + rb * 8 + r,
                                    pl.ds(cbase + o, L)]
                        idx = lanes16 + (
                            obase + cbase * N_LATENT + o * N_LATENT + d)
                        plsc.store_scatter(out_v, [idx], v)
                return c2

            lax.fori_loop(0, KB, blk, 0)
            pltpu.async_copy(
                out_v.at[pl.ds(obase, SUPO)],
                dst_hbm.at[pl.ds((lo + j) * SUPO, SUPO)],
                sem_out)
            return carry

        lax.fori_loop(0, nm, sup, 0)
        lax.fori_loop(
            0, jnp.minimum(nm, 2),
            lambda i, c: (pltpu.make_async_copy(
                out_v.at[pl.ds(0, SUPO)], dst_hbm.at[pl.ds(0, SUPO)],
                sem_out).wait(), c)[1],
            0)

    do_table(tht_hbm, thf_hbm)
    do_table(xt_hbm, xf_hbm)

    @pl.when(wid == NW - 1)
    def _():
        for tail_hbm, dst_hbm in ((th_tail_hbm, thf_hbm),
                                  (x_tail_hbm, xf_hbm)):
            sl = pl.ds(0, TAIL * N_LATENT)
            pltpu.sync_copy(tail_hbm, out_v.at[sl])
            pltpu.sync_copy(
                out_v.at[sl],
                dst_hbm.at[pl.ds(NBLK * 2048, TAIL * N_LATENT)])


def _body(theta_hbm, x_hbm, tgt_hbm, uidx_hbm, iidx_hbm,
          pred_hbm, loss_hbm,
          uidx_v, iidx_v, tgt_v, th_v, x_v, pred_v, stage_v,
          lpart_v, lred_v, loss_shared, sem_idx, sem_rows, sem_out):
    cid = lax.axis_index("c")
    sid = lax.axis_index("s")
    wid = cid * NS + sid
    base_w = wid * PER_W

    lanes_x16 = lax.iota(jnp.int32, L) * L

    def issue_idx(k, off):
        start = base_w + k * CHUNK
        sl = pl.ds(off, CHUNK)
        pltpu.async_copy(uidx_hbm.at[pl.ds(start, CHUNK)], uidx_v.at[sl],
                         sem_idx)
        pltpu.async_copy(iidx_hbm.at[pl.ds(start, CHUNK)], iidx_v.at[sl],
                         sem_idx)
        pltpu.async_copy(tgt_hbm.at[pl.ds(start, CHUNK)], tgt_v.at[sl],
                         sem_idx)

    def wait_idx():
        for src, ref in ((uidx_hbm, uidx_v), (iidx_hbm, iidx_v),
                         (tgt_hbm, tgt_v)):
            pltpu.make_async_copy(
                src.at[pl.ds(0, CHUNK)], ref.at[pl.ds(0, CHUNK)],
                sem_idx).wait()

    def issue_gathers(off):
        for j in range(N_SUB):
            sl = pl.ds(off + j * SUB, SUB)
            pltpu.async_copy(theta_hbm.at[uidx_v.at[sl]], th_v.at[sl],
                             sem_rows)
            pltpu.async_copy(x_hbm.at[iidx_v.at[sl]], x_v.at[sl],
                             sem_rows)

    def wait_gathers():
        for ref in (th_v, x_v):
            pltpu.make_async_copy(
                theta_hbm.at[pl.ds(0, CHUNK)], ref.at[pl.ds(0, CHUNK)],
                sem_rows).wait()

    issue_idx(0, 0)
    wait_idx()
    issue_gathers(0)
    issue_idx(1, CHUNK)

    def chunk_body(k, loss_acc):
        off = (k % 2) * CHUNK
        start = base_w + k * CHUNK
        wait_gathers()

        @pl.when(k + 1 < N_CHUNKS)
        def _():
            wait_idx()
            issue_gathers((k + 1) % 2 * CHUNK)

        @pl.when(k >= 2)
        def _():
            pltpu.make_async_copy(
                pred_v.at[pl.ds(0, CHUNK)], pred_hbm.at[pl.ds(0, CHUNK)],
                sem_out).wait()

        def group_body(g, acc):
            rbase = off + g * L
            sbase = (g % 2) * (L * L)
            for i in range(L):
                r = rbase + i
                prod = th_v[r] * x_v[r]
                plsc.store_scatter(stage_v, [lanes_x16 + (sbase + i)], prod)
            p = stage_v[pl.ds(sbase, L)]
            for d in range(1, L):
                p = p + stage_v[pl.ds(sbase + d * L, L)]
            pred_v[pl.ds(rbase, L)] = p
            diff = p - tgt_v[pl.ds(rbase, L)]
            return acc + diff * diff

        loss_acc = lax.fori_loop(0, N_GROUPS, group_body, loss_acc)
        pltpu.async_copy(pred_v.at[pl.ds(off, CHUNK)],
                         pred_hbm.at[pl.ds(start, CHUNK)], sem_out)

        @pl.when(k + 2 < N_CHUNKS)
        def _():
            issue_idx(k + 2, off)

        return loss_acc

    loss_acc = lax.fori_loop(
        0, N_CHUNKS, chunk_body, jnp.zeros((L,), jnp.float32))

    for _ in range(2):
        pltpu.make_async_copy(
            pred_v.at[pl.ds(0, CHUNK)], pred_hbm.at[pl.ds(0, CHUNK)],
            sem_out).wait()

    lpart_v[...] = loss_acc
    pltpu.sync_copy(lpart_v, loss_shared.at[sid])
    plsc.subcore_barrier()

    @pl.when(sid == 0)
    def _():
        pltpu.sync_copy(loss_shared, lred_v)
        tot = lred_v[0]
        for s in range(1, NS):
            tot = tot + lred_v[s]
        scalar = jnp.sum(tot) * (1.0 / float(N_REVIEWS))
        lpart_v[...] = jnp.broadcast_to(scalar, (L,))
        pltpu.sync_copy(lpart_v, loss_hbm.at[cid])


@jax.jit
def _mf_forward(theta, X, targets, uidx, iidx):
    mesh = plsc.VectorSubcoreMesh(
        core_axis_name="c", subcore_axis_name="s",
        num_cores=NC, num_subcores=NS)
    untangle = pl.kernel(
        _untangle_body,
        out_type=(
            jax.ShapeDtypeStruct((N_USERS * N_LATENT,), jnp.float32),
            jax.ShapeDtypeStruct((N_ITEMS * N_LATENT,), jnp.float32),
        ),
        mesh=mesh,
        scratch_types=[
            pltpu.VMEM((2 * L, SUPW), jnp.float32),
            pltpu.VMEM((2 * SUPO,), jnp.float32),
            pltpu.SemaphoreType.DMA,
            pltpu.SemaphoreType.DMA,
        ],
        compiler_params=pltpu.CompilerParams(
            needs_layout_passes=False, use_tc_tiling_on_sc=True),
    )
    th_tail = theta[NBLK * 128:].reshape(-1)
    x_tail = X[NBLK * 128:].reshape(-1)
    th_flat, x_flat = untangle(theta.T, X.T, th_tail, x_tail)
    theta_rm = th_flat.reshape(N_USERS, N_LATENT)
    x_rm = x_flat.reshape(N_ITEMS, N_LATENT)
    f = pl.kernel(
        _body,
        out_type=(
            jax.ShapeDtypeStruct((N_REVIEWS,), jnp.float32),
            jax.ShapeDtypeStruct((NC, L), jnp.float32),
        ),
        mesh=mesh,
        scratch_types=[
            pltpu.VMEM((2 * CHUNK,), jnp.int32),
            pltpu.VMEM((2 * CHUNK,), jnp.int32),
            pltpu.VMEM((2 * CHUNK,), jnp.float32),
            pltpu.VMEM((2 * CHUNK, N_LATENT), jnp.float32),
            pltpu.VMEM((2 * CHUNK, N_LATENT), jnp.float32),
            pltpu.VMEM((2 * CHUNK,), jnp.float32),
            pltpu.VMEM((2 * L * L,), jnp.float32),
            pltpu.VMEM((L,), jnp.float32),
            pltpu.VMEM((NS, L), jnp.float32),
            pltpu.VMEM_SHARED((NS, L), jnp.float32),
            pltpu.SemaphoreType.DMA,
            pltpu.SemaphoreType.DMA,
            pltpu.SemaphoreType.DMA,
        ],
        compiler_params=pltpu.CompilerParams(
            needs_layout_passes=False, use_tc_tiling_on_sc=False),
    )
    return f(theta_rm, x_rm, targets, uidx, iidx)


def kernel(theta, X, targets, user_indices, item_indices):
    uidx = user_indices.astype(jnp.int32)
    iidx = item_indices.astype(jnp.int32)
    pred, loss_parts = _mf_forward(theta, X, targets, uidx, iidx)
    loss = loss_parts[0, 0] + loss_parts[1, 0]
    return (pred, loss)
